# baseline (speedup 1.0000x reference)
import jax
import jax.numpy as jnp
from jax.experimental import pallas as pl

N, D, H, O, E = 4096, 1024, 2048, 1024, 3
BN = 256


def _moe_kernel(x_ref, gw1_ref, gb1_ref, gw2_ref, gb2_ref, gw3_ref, gb3_ref,
                ew1_ref, eb1_ref, ew2_ref, eb2_ref, out_ref, gates_ref):
    xb = x_ref[...]

    h1 = jax.nn.relu(
        jnp.dot(xb, gw1_ref[...], preferred_element_type=jnp.float32)
        + gb1_ref[...])
    h2 = jax.nn.relu(
        jnp.dot(h1, gw2_ref[...], preferred_element_type=jnp.float32)
        + gb2_ref[...])
    logits = jnp.dot(h2, gw3_ref[...], preferred_element_type=jnp.float32) \
        + gb3_ref[...]
    gates = jax.nn.softmax(logits, axis=-1)
    gates_ref[...] = gates

    g0, g1, g2 = gates[:, 0], gates[:, 1], gates[:, 2]
    drop2 = (g2 <= g0) & (g2 <= g1)
    drop1 = (~drop2) & (g1 <= g0) & (g1 <= g2)
    drop0 = (~drop2) & (~drop1)
    gmin = jnp.where(drop2, g2, jnp.where(drop1, g1, g0))
    denom = (g0 + g1 + g2) - gmin
    w0 = jnp.where(drop0, 0.0, g0) / denom
    w1 = jnp.where(drop1, 0.0, g1) / denom
    w2 = jnp.where(drop2, 0.0, g2) / denom

    xb16 = xb.astype(jnp.bfloat16)
    HC = H // 2
    acc = jnp.zeros((xb.shape[0], O), jnp.float32)
    for e, we in ((0, w0), (1, w1), (2, w2)):
        o = eb2_ref[e].astype(jnp.float32)[None, :]
        for k in range(2):
            h = jax.nn.relu(
                jnp.dot(xb16, ew1_ref[e][:, k * HC:(k + 1) * HC],
                        preferred_element_type=jnp.float32)
                + eb1_ref[e][k * HC:(k + 1) * HC])
            o = o + jnp.dot(h.astype(jnp.bfloat16),
                            ew2_ref[e][k * HC:(k + 1) * HC, :],
                            preferred_element_type=jnp.float32)
        acc = acc + we[:, None] * o
    out_ref[...] = acc


def kernel(x, gw1, gb1, gw2, gb2, gw3, gb3, ew1, eb1, ew2, eb2):
    ew1 = ew1.astype(jnp.bfloat16)
    ew2 = ew2.astype(jnp.bfloat16)
    eb1 = eb1.astype(jnp.bfloat16)
    gb1 = gb1.reshape(1, -1)
    gb2 = gb2.reshape(1, -1)
    gb3 = gb3.reshape(1, -1)

    grid = (N // BN,)
    full = lambda i: (0, 0)
    full3 = lambda i: (0, 0, 0)
    out, gates = pl.pallas_call(
        _moe_kernel,
        grid=grid,
        in_specs=[
            pl.BlockSpec((BN, D), lambda i: (i, 0)),
            pl.BlockSpec((D, 256), full),
            pl.BlockSpec((1, 256), full),
            pl.BlockSpec((256, 128), full),
            pl.BlockSpec((1, 128), full),
            pl.BlockSpec((128, E), full),
            pl.BlockSpec((1, E), full),
            pl.BlockSpec((E, D, H), full3),
            pl.BlockSpec((E, H), full),
            pl.BlockSpec((E, H, O), full3),
            pl.BlockSpec((E, O), full),
        ],
        out_specs=[
            pl.BlockSpec((BN, O), lambda i: (i, 0)),
            pl.BlockSpec((BN, E), lambda i: (i, 0)),
        ],
        out_shape=[
            jax.ShapeDtypeStruct((N, O), jnp.float32),
            jax.ShapeDtypeStruct((N, E), jnp.float32),
        ],
    )(x, gw1, gb1, gw2, gb2, gw3, gb3, ew1, eb1, ew2, eb2)
    return out, gates


# speedup vs baseline: 1.0132x; 1.0132x over previous
import jax
import jax.numpy as jnp
from jax.experimental import pallas as pl

N, D, H, O, E = 4096, 1024, 2048, 1024, 3
BN = 256


def _moe_kernel(x_ref, gw1_ref, gb1_ref, gw2_ref, gb2_ref, gw3_ref, gb3_ref,
                ew1_ref, eb1_ref, ew2_ref, eb2_ref, out_ref, gates_ref):
    xb = x_ref[...]

    h1 = jax.nn.relu(
        jnp.dot(xb, gw1_ref[...], preferred_element_type=jnp.float32)
        + gb1_ref[...])
    h2 = jax.nn.relu(
        jnp.dot(h1, gw2_ref[...], preferred_element_type=jnp.float32)
        + gb2_ref[...])
    logits = jnp.dot(h2, gw3_ref[...], preferred_element_type=jnp.float32) \
        + gb3_ref[...]
    gates = jax.nn.softmax(logits, axis=-1)
    gates_ref[...] = gates

    g0, g1, g2 = gates[:, 0], gates[:, 1], gates[:, 2]
    drop2 = (g2 <= g0) & (g2 <= g1)
    drop1 = (~drop2) & (g1 <= g0) & (g1 <= g2)
    drop0 = (~drop2) & (~drop1)
    gmin = jnp.where(drop2, g2, jnp.where(drop1, g1, g0))
    denom = (g0 + g1 + g2) - gmin
    w0 = jnp.where(drop0, 0.0, g0) / denom
    w1 = jnp.where(drop1, 0.0, g1) / denom
    w2 = jnp.where(drop2, 0.0, g2) / denom

    xb16 = xb.astype(jnp.bfloat16)
    acc = jnp.zeros((xb.shape[0], O), jnp.float32)
    for e, we in ((0, w0), (1, w1), (2, w2)):
        h = jax.nn.relu(
            jnp.dot(xb16, ew1_ref[e], preferred_element_type=jnp.float32)
            + eb1_ref[e])
        o = jnp.dot(h.astype(jnp.bfloat16), ew2_ref[e],
                    preferred_element_type=jnp.float32) + eb2_ref[e]
        acc = acc + we[:, None] * o
    out_ref[...] = acc


def kernel(x, gw1, gb1, gw2, gb2, gw3, gb3, ew1, eb1, ew2, eb2):
    ew1 = ew1.astype(jnp.bfloat16)
    ew2 = ew2.astype(jnp.bfloat16)
    gb1 = gb1.reshape(1, -1)
    gb2 = gb2.reshape(1, -1)
    gb3 = gb3.reshape(1, -1)

    grid = (N // BN,)
    full = lambda i: (0, 0)
    full3 = lambda i: (0, 0, 0)
    out, gates = pl.pallas_call(
        _moe_kernel,
        grid=grid,
        in_specs=[
            pl.BlockSpec((BN, D), lambda i: (i, 0)),
            pl.BlockSpec((D, 256), full),
            pl.BlockSpec((1, 256), full),
            pl.BlockSpec((256, 128), full),
            pl.BlockSpec((1, 128), full),
            pl.BlockSpec((128, E), full),
            pl.BlockSpec((1, E), full),
            pl.BlockSpec((E, D, H), full3),
            pl.BlockSpec((E, H), full),
            pl.BlockSpec((E, H, O), full3),
            pl.BlockSpec((E, O), full),
        ],
        out_specs=[
            pl.BlockSpec((BN, O), lambda i: (i, 0)),
            pl.BlockSpec((BN, E), lambda i: (i, 0)),
        ],
        out_shape=[
            jax.ShapeDtypeStruct((N, O), jnp.float32),
            jax.ShapeDtypeStruct((N, E), jnp.float32),
        ],
    )(x, gw1, gb1, gw2, gb2, gw3, gb3, ew1, eb1, ew2, eb2)
    return out, gates
